# baseline (speedup 1.0000x reference)
import jax
import jax.numpy as jnp
import numpy as np
from jax.experimental import pallas as pl
from jax.experimental.pallas import tpu as pltpu

N = 2048
N_RBF = 16
N_HIDDEN = 64
CUTOFF = 5.0
R_MIN = 0.5
BLK = 256

_CENTERS = np.linspace(R_MIN, CUTOFF, N_RBF).astype(np.float32)
_ETA = np.float32(0.5 * (CUTOFF - R_MIN) / N_RBF)
_INV2ETA2 = np.float32(1.0 / (2.0 * _ETA * _ETA))
_PI = np.float32(np.pi)
_SINC0 = np.float32(-0.5 * np.pi)
_SINC1 = np.float32(0.5 * np.pi ** 3 / 6.0)
_SINC2 = np.float32(-0.5 * np.pi ** 5 / 120.0)
_SINC3 = np.float32(0.5 * np.pi ** 7 / 5040.0)
_SINC4 = np.float32(-0.5 * np.pi ** 9 / 362880.0)
_SINC5 = np.float32(0.5 * np.pi ** 11 / 39916800.0)


def _fused_kernel(pos_blk_ref, pos_t_ref, w1_ref, b1_ref, w2_ref, b2_ref,
                  w3_ref, b3_ref, out_ref):
    i = pl.program_id(0)
    xi = pos_blk_ref[:, 0:1]
    yi = pos_blk_ref[:, 1:2]
    zi = pos_blk_ref[:, 2:3]
    xj = pos_t_ref[0:1, :]
    yj = pos_t_ref[1:2, :]
    zj = pos_t_ref[2:3, :]
    dx = xj - xi
    dy = yj - yi
    dz = zj - zi
    sq = dx * dx + dy * dy + dz * dz
    dist = jnp.sqrt(sq)
    x = dist * np.float32(1.0 / CUTOFF)
    z = x - np.float32(0.5)
    s = z * z
    q = _SINC5
    q = q * s + _SINC4
    q = q * s + _SINC3
    q = q * s + _SINC2
    q = q * s + _SINC1
    q = q * s + _SINC0
    smooth = jnp.maximum(np.float32(0.5) + z * q, 0.0)
    w = jnp.where((sq > 0.0) & (dist < CUTOFF), smooth, 0.0)
    logw = jnp.log(w)
    ds = dist * np.float32(np.sqrt(_INV2ETA2))
    l2e = np.float32(np.log2(np.e))
    cm = np.float32(0.5 * (_CENTERS[0] + _CENTERS[-1]) * np.sqrt(_INV2ETA2))
    u = ds - cm
    al = l2e * (logw - u * u)
    ul = (l2e + l2e) * u
    ones_col = jnp.ones((N, 1), jnp.float32)
    cols = []
    for k in range(N_RBF):
        bk = np.float32(_CENTERS[k] * np.sqrt(_INV2ETA2) - cm)
        bkl = np.float32(bk * np.log2(np.e))
        r = jnp.exp2((ul - bkl) * bk + al)
        cols.append(jax.lax.dot(r, ones_col,
                                preferred_element_type=jnp.float32))
    features = jnp.concatenate(cols, axis=1)
    h = features @ w1_ref[...] + b1_ref[...]
    h = jax.nn.silu(h)
    h = h @ w2_ref[...] + b2_ref[...]
    h = jax.nn.silu(h)
    atomic_e = h @ w3_ref[...] + b3_ref[...]
    out_ref[...] = jnp.sum(atomic_e).reshape(1, 1, 1)


def kernel(positions, W1, b1, W2, b2, W3, b3):
    pos_t = positions.T
    b1r = b1.reshape(1, N_HIDDEN)
    b2r = b2.reshape(1, N_HIDDEN)
    b3r = b3.reshape(1, 1)
    out = pl.pallas_call(
        _fused_kernel,
        grid=(N // BLK,),
        in_specs=[
            pl.BlockSpec((BLK, 3), lambda i: (i, 0)),
            pl.BlockSpec((3, N), lambda i: (0, 0)),
            pl.BlockSpec((N_RBF, N_HIDDEN), lambda i: (0, 0)),
            pl.BlockSpec((1, N_HIDDEN), lambda i: (0, 0)),
            pl.BlockSpec((N_HIDDEN, N_HIDDEN), lambda i: (0, 0)),
            pl.BlockSpec((1, N_HIDDEN), lambda i: (0, 0)),
            pl.BlockSpec((N_HIDDEN, 1), lambda i: (0, 0)),
            pl.BlockSpec((1, 1), lambda i: (0, 0)),
        ],
        out_specs=pl.BlockSpec((1, 1, 1), lambda i: (i, 0, 0)),
        out_shape=jax.ShapeDtypeStruct((N // BLK, 1, 1), jnp.float32),
        compiler_params=pltpu.CompilerParams(
            dimension_semantics=("parallel",)),
    )(positions, pos_t, W1, b1r, W2, b2r, W3, b3r)
    return jnp.sum(out)


# speedup vs baseline: 1.1144x; 1.1144x over previous
import jax
import jax.numpy as jnp
import numpy as np
from jax.experimental import pallas as pl
from jax.experimental.pallas import tpu as pltpu

N = 2048
N_RBF = 16
N_HIDDEN = 64
CUTOFF = 5.0
R_MIN = 0.5
BLK = 512
NBLK = N // BLK

_CENTERS = np.linspace(R_MIN, CUTOFF, N_RBF).astype(np.float32)
_ETA = np.float32(0.5 * (CUTOFF - R_MIN) / N_RBF)
_INV2ETA2 = np.float32(1.0 / (2.0 * _ETA * _ETA))
_SQB = np.float32(np.sqrt(_INV2ETA2))
_SINC0 = np.float32(-0.5 * np.pi)
_SINC1 = np.float32(0.5 * np.pi ** 3 / 6.0)
_SINC2 = np.float32(-0.5 * np.pi ** 5 / 120.0)
_SINC3 = np.float32(0.5 * np.pi ** 7 / 5040.0)
_SINC4 = np.float32(-0.5 * np.pi ** 9 / 362880.0)
_SINC5 = np.float32(0.5 * np.pi ** 11 / 39916800.0)


def _fused_kernel(pos_i_ref, pos_j_ref, w1_ref, b1_ref, w2_ref, b2_ref,
                  w3_ref, b3_ref, eye_ref, out_ref, f_scr, g_scr):
    bi = pl.program_id(0)
    bj = pl.program_id(1)

    @pl.when((bi == 0) & (bj == 0))
    def _init():
        f_scr[...] = jnp.zeros((N, N_RBF), jnp.float32)
        g_scr[...] = jnp.zeros((N_RBF, N), jnp.float32)

    @pl.when(bj >= bi)
    def _tile():
        xi = pos_i_ref[:, 0:1]
        yi = pos_i_ref[:, 1:2]
        zi = pos_i_ref[:, 2:3]
        xj2 = pos_j_ref[0:1, :]
        yj2 = pos_j_ref[1:2, :]
        zj2 = pos_j_ref[2:3, :]
        ni = xi * xi + yi * yi + zi * zi
        nj = (xj2 * xj2 + yj2 * yj2 + zj2 * zj2) * np.float32(0.25)
        g2 = jax.lax.dot(pos_i_ref[...], pos_j_ref[...],
                         preferred_element_type=jnp.float32)
        sq = jnp.maximum((ni + nj) + g2, 0.0)
        row_ids = bi * BLK + jax.lax.broadcasted_iota(jnp.int32, (BLK, 1), 0)
        col_ids = bj * BLK + jax.lax.broadcasted_iota(jnp.int32, (1, BLK), 1)
        neq = row_ids != col_ids
        dist = jnp.sqrt(sq)
        x = dist * np.float32(1.0 / CUTOFF)
        z = x - np.float32(0.5)
        s = z * z
        q = _SINC5
        q = q * s + _SINC4
        q = q * s + _SINC3
        q = q * s + _SINC2
        q = q * s + _SINC1
        q = q * s + _SINC0
        smooth = jnp.maximum(np.float32(0.5) + z * q, 0.0)
        w = jnp.where(neq & (dist < CUTOFF), smooth, 0.0)
        logw = jnp.log(w)
        ds = dist * _SQB
        l2e = np.float32(np.log2(np.e))
        cm = np.float32(0.5 * (_CENTERS[0] + _CENTERS[-1]) * _SQB)
        u = ds - cm
        al = l2e * (logw - u * u)
        ul = (l2e + l2e) * u
        ones_col = jnp.ones((BLK, 1), jnp.float32)
        row_cols = []
        col_rows = []
        for k in range(N_RBF):
            bk = np.float32(_CENTERS[k] * _SQB - cm)
            bkl = np.float32(bk * np.log2(np.e))
            e = jnp.exp2((ul - bkl) * bk + al)
            row_cols.append(jax.lax.dot(e, ones_col,
                                        preferred_element_type=jnp.float32))
            col_rows.append(jax.lax.dot_general(
                ones_col, e, (((0,), (0,)), ((), ())),
                preferred_element_type=jnp.float32))
        feats_i = jnp.concatenate(row_cols, axis=1)
        f_scr[pl.ds(bi * BLK, BLK), :] += feats_i

        @pl.when(bj > bi)
        def _colside():
            feats_j = jnp.concatenate(col_rows, axis=0)
            g_scr[:, pl.ds(bj * BLK, BLK)] += feats_j

    @pl.when((bi == NBLK - 1) & (bj == NBLK - 1))
    def _finish():
        gt = jax.lax.dot_general(g_scr[...], eye_ref[...],
                                 (((0,), (0,)), ((), ())),
                                 preferred_element_type=jnp.float32)
        features = f_scr[...] + gt
        h = features @ w1_ref[...] + b1_ref[...]
        h = h * jax.nn.sigmoid(h)
        h = h @ w2_ref[...] + b2_ref[...]
        h = h * jax.nn.sigmoid(h)
        atomic_e = h @ w3_ref[...] + b3_ref[...]
        out_ref[...] = jnp.sum(atomic_e).reshape(1, 1)


def kernel(positions, W1, b1, W2, b2, W3, b3):
    pos_t = (positions * np.float32(-2.0)).T
    b1r = b1.reshape(1, N_HIDDEN)
    b2r = b2.reshape(1, N_HIDDEN)
    b3r = b3.reshape(1, 1)
    eye = jnp.eye(N_RBF, dtype=jnp.float32)
    out = pl.pallas_call(
        _fused_kernel,
        grid=(NBLK, NBLK),
        in_specs=[
            pl.BlockSpec((BLK, 3), lambda bi, bj: (bi, 0)),
            pl.BlockSpec((3, BLK), lambda bi, bj: (0, bj)),
            pl.BlockSpec((N_RBF, N_HIDDEN), lambda bi, bj: (0, 0)),
            pl.BlockSpec((1, N_HIDDEN), lambda bi, bj: (0, 0)),
            pl.BlockSpec((N_HIDDEN, N_HIDDEN), lambda bi, bj: (0, 0)),
            pl.BlockSpec((1, N_HIDDEN), lambda bi, bj: (0, 0)),
            pl.BlockSpec((N_HIDDEN, 1), lambda bi, bj: (0, 0)),
            pl.BlockSpec((1, 1), lambda bi, bj: (0, 0)),
            pl.BlockSpec((N_RBF, N_RBF), lambda bi, bj: (0, 0)),
        ],
        out_specs=pl.BlockSpec((1, 1), lambda bi, bj: (0, 0)),
        out_shape=jax.ShapeDtypeStruct((1, 1), jnp.float32),
        scratch_shapes=[
            pltpu.VMEM((N, N_RBF), jnp.float32),
            pltpu.VMEM((N_RBF, N), jnp.float32),
        ],
    )(positions, pos_t, W1, b1r, W2, b2r, W3, b3r, eye)
    return out[0, 0]


# speedup vs baseline: 1.1156x; 1.0011x over previous
import jax
import jax.numpy as jnp
import numpy as np
from jax.experimental import pallas as pl
from jax.experimental.pallas import tpu as pltpu

N = 2048
N_RBF = 16
N_HIDDEN = 64
CUTOFF = 5.0
R_MIN = 0.5
BLK = 512
NBLK = N // BLK

_CENTERS = np.linspace(R_MIN, CUTOFF, N_RBF).astype(np.float32)
_ETA = np.float32(0.5 * (CUTOFF - R_MIN) / N_RBF)
_INV2ETA2 = np.float32(1.0 / (2.0 * _ETA * _ETA))
_SQB = np.float32(np.sqrt(_INV2ETA2))
_SINC0 = np.float32(-0.5 * np.pi)
_SINC1 = np.float32(0.5 * np.pi ** 3 / 6.0)
_SINC2 = np.float32(-0.5 * np.pi ** 5 / 120.0)
_SINC3 = np.float32(0.5 * np.pi ** 7 / 5040.0)
_SINC4 = np.float32(-0.5 * np.pi ** 9 / 362880.0)
_SINC5 = np.float32(0.5 * np.pi ** 11 / 39916800.0)


def _fused_kernel(pos_i_ref, pos_j_ref, w1_ref, b1_ref, w2_ref, b2_ref,
                  w3_ref, b3_ref, eye_ref, out_ref, f_scr, g_scr):
    bi = pl.program_id(0)
    bj = pl.program_id(1)

    @pl.when((bi == 0) & (bj == 0))
    def _init():
        f_scr[...] = jnp.zeros((N, N_RBF), jnp.float32)
        g_scr[...] = jnp.zeros((N_RBF, N), jnp.float32)

    @pl.when(bj >= bi)
    def _tile():
        xi = pos_i_ref[:, 0:1]
        yi = pos_i_ref[:, 1:2]
        zi = pos_i_ref[:, 2:3]
        xj = pos_j_ref[0:1, :]
        yj = pos_j_ref[1:2, :]
        zj = pos_j_ref[2:3, :]
        dx = xj - xi
        dy = yj - yi
        dz = zj - zi
        sq = dx * dx + dy * dy + dz * dz
        dist = jnp.sqrt(sq)
        x = dist * np.float32(1.0 / CUTOFF)
        z = x - np.float32(0.5)
        s = z * z
        q = _SINC5
        q = q * s + _SINC4
        q = q * s + _SINC3
        q = q * s + _SINC2
        q = q * s + _SINC1
        q = q * s + _SINC0
        smooth = jnp.maximum(np.float32(0.5) + z * q, 0.0)
        w = jnp.where((sq > 0.0) & (dist < CUTOFF), smooth, 0.0)
        logw = jnp.log(w)
        ds = dist * _SQB
        l2e = np.float32(np.log2(np.e))
        cm = np.float32(0.5 * (_CENTERS[0] + _CENTERS[-1]) * _SQB)
        u = ds - cm
        al = l2e * (logw - u * u)
        ul = (l2e + l2e) * u
        ones_col = jnp.ones((BLK, 1), jnp.float32)
        bks = [np.float32(c * _SQB - cm) for c in _CENTERS]
        db = np.float32(bks[1] - bks[0])
        d_step = ul * db
        row_cols = []
        col_rows = []
        arg = None
        for k in range(N_RBF):
            bk = bks[k]
            bkl = np.float32(bk * np.log2(np.e))
            if k % 4 == 0:
                arg = (ul - bkl) * bk + al
            else:
                bprev = bks[k - 1]
                s_k = np.float32((bk * bk - bprev * bprev) * np.log2(np.e))
                arg = (arg + d_step) - s_k
            e = jnp.exp2(arg)
            row_cols.append(jax.lax.dot(e, ones_col,
                                        preferred_element_type=jnp.float32))
            col_rows.append(jax.lax.dot_general(
                ones_col, e, (((0,), (0,)), ((), ())),
                preferred_element_type=jnp.float32))
        feats_i = jnp.concatenate(row_cols, axis=1)
        f_scr[pl.ds(bi * BLK, BLK), :] += feats_i

        @pl.when(bj > bi)
        def _colside():
            feats_j = jnp.concatenate(col_rows, axis=0)
            g_scr[:, pl.ds(bj * BLK, BLK)] += feats_j

    @pl.when((bi == NBLK - 1) & (bj == NBLK - 1))
    def _finish():
        gt = jax.lax.dot_general(g_scr[...], eye_ref[...],
                                 (((0,), (0,)), ((), ())),
                                 preferred_element_type=jnp.float32)
        features = f_scr[...] + gt
        h = features @ w1_ref[...] + b1_ref[...]
        h = h * jax.nn.sigmoid(h)
        h = h @ w2_ref[...] + b2_ref[...]
        h = h * jax.nn.sigmoid(h)
        atomic_e = h @ w3_ref[...] + b3_ref[...]
        out_ref[...] = jnp.sum(atomic_e).reshape(1, 1)


def kernel(positions, W1, b1, W2, b2, W3, b3):
    pos_t = positions.T
    b1r = b1.reshape(1, N_HIDDEN)
    b2r = b2.reshape(1, N_HIDDEN)
    b3r = b3.reshape(1, 1)
    eye = jnp.eye(N_RBF, dtype=jnp.float32)
    out = pl.pallas_call(
        _fused_kernel,
        grid=(NBLK, NBLK),
        in_specs=[
            pl.BlockSpec((BLK, 3), lambda bi, bj: (bi, 0)),
            pl.BlockSpec((3, BLK), lambda bi, bj: (0, bj)),
            pl.BlockSpec((N_RBF, N_HIDDEN), lambda bi, bj: (0, 0)),
            pl.BlockSpec((1, N_HIDDEN), lambda bi, bj: (0, 0)),
            pl.BlockSpec((N_HIDDEN, N_HIDDEN), lambda bi, bj: (0, 0)),
            pl.BlockSpec((1, N_HIDDEN), lambda bi, bj: (0, 0)),
            pl.BlockSpec((N_HIDDEN, 1), lambda bi, bj: (0, 0)),
            pl.BlockSpec((1, 1), lambda bi, bj: (0, 0)),
            pl.BlockSpec((N_RBF, N_RBF), lambda bi, bj: (0, 0)),
        ],
        out_specs=pl.BlockSpec((1, 1), lambda bi, bj: (0, 0)),
        out_shape=jax.ShapeDtypeStruct((1, 1), jnp.float32),
        scratch_shapes=[
            pltpu.VMEM((N, N_RBF), jnp.float32),
            pltpu.VMEM((N_RBF, N), jnp.float32),
        ],
    )(positions, pos_t, W1, b1r, W2, b2r, W3, b3r, eye)
    return out[0, 0]


# speedup vs baseline: 1.2896x; 1.1559x over previous
import jax
import jax.numpy as jnp
import numpy as np
from jax.experimental import pallas as pl
from jax.experimental.pallas import tpu as pltpu

N = 2048
N_RBF = 16
N_HIDDEN = 64
CUTOFF = 5.0
R_MIN = 0.5
BLK = 512
NBLK = N // BLK

_CENTERS = np.linspace(R_MIN, CUTOFF, N_RBF).astype(np.float32)
_ETA = np.float32(0.5 * (CUTOFF - R_MIN) / N_RBF)
_INV2ETA2 = np.float32(1.0 / (2.0 * _ETA * _ETA))
_SQB = np.float32(np.sqrt(_INV2ETA2))
_SINC0 = np.float32(-0.5 * np.pi)
_SINC1 = np.float32(0.5 * np.pi ** 3 / 6.0)
_SINC2 = np.float32(-0.5 * np.pi ** 5 / 120.0)
_SINC3 = np.float32(0.5 * np.pi ** 7 / 5040.0)
_SINC4 = np.float32(-0.5 * np.pi ** 9 / 362880.0)
_SINC5 = np.float32(0.5 * np.pi ** 11 / 39916800.0)


def _fused_kernel(pos_i_ref, pos_j_ref, w1_ref, b1_ref, w2_ref, b2_ref,
                  w3_ref, b3_ref, eye_ref, out_ref, f_scr, g_scr):
    bi = pl.program_id(0)
    bj = pl.program_id(1)

    @pl.when((bi == 0) & (bj == 0))
    def _init():
        f_scr[...] = jnp.zeros((N, N_RBF), jnp.float32)
        g_scr[...] = jnp.zeros((N_RBF, N), jnp.float32)

    @pl.when(bj >= bi)
    def _tile():
        xi = pos_i_ref[:, 0:1]
        yi = pos_i_ref[:, 1:2]
        zi = pos_i_ref[:, 2:3]
        xj = pos_j_ref[0:1, :]
        yj = pos_j_ref[1:2, :]
        zj = pos_j_ref[2:3, :]
        dx = xj - xi
        dy = yj - yi
        dz = zj - zi
        sq = dx * dx + dy * dy + dz * dz
        dist = jnp.sqrt(sq)
        x = dist * np.float32(1.0 / CUTOFF)
        z = x - np.float32(0.5)
        s = z * z
        q = _SINC5
        q = q * s + _SINC4
        q = q * s + _SINC3
        q = q * s + _SINC2
        q = q * s + _SINC1
        q = q * s + _SINC0
        smooth = jnp.maximum(np.float32(0.5) + z * q, 0.0)
        w = jnp.where((sq > 0.0) & (dist < CUTOFF), smooth, 0.0)
        logw = jnp.log(w)
        ds = dist * _SQB
        l2e = np.float32(np.log2(np.e))
        cm = np.float32(0.5 * (_CENTERS[0] + _CENTERS[-1]) * _SQB)
        u = ds - cm
        al = l2e * (logw - u * u)
        ul = (l2e + l2e) * u
        ones_col = jnp.ones((BLK, 1), jnp.float32)
        row_cols = []
        col_rows = []
        for k in range(N_RBF):
            bk = np.float32(_CENTERS[k] * _SQB - cm)
            bkl = np.float32(bk * np.log2(np.e))
            e = jnp.exp2((ul - bkl) * bk + al)
            row_cols.append(jax.lax.dot(e, ones_col,
                                        preferred_element_type=jnp.float32))
            col_rows.append(jnp.sum(e, axis=0, keepdims=True))
        feats_i = jnp.concatenate(row_cols, axis=1)
        f_scr[pl.ds(bi * BLK, BLK), :] += feats_i

        @pl.when(bj > bi)
        def _colside():
            feats_j = jnp.concatenate(col_rows, axis=0)
            g_scr[:, pl.ds(bj * BLK, BLK)] += feats_j

    @pl.when((bi == NBLK - 1) & (bj == NBLK - 1))
    def _finish():
        gt = jax.lax.dot_general(g_scr[...], eye_ref[...],
                                 (((0,), (0,)), ((), ())),
                                 preferred_element_type=jnp.float32)
        features = f_scr[...] + gt
        h = features @ w1_ref[...] + b1_ref[...]
        h = h * jax.nn.sigmoid(h)
        h = h @ w2_ref[...] + b2_ref[...]
        h = h * jax.nn.sigmoid(h)
        atomic_e = h @ w3_ref[...] + b3_ref[...]
        out_ref[...] = jnp.sum(atomic_e).reshape(1, 1)


def kernel(positions, W1, b1, W2, b2, W3, b3):
    pos_t = positions.T
    b1r = b1.reshape(1, N_HIDDEN)
    b2r = b2.reshape(1, N_HIDDEN)
    b3r = b3.reshape(1, 1)
    eye = jnp.eye(N_RBF, dtype=jnp.float32)
    out = pl.pallas_call(
        _fused_kernel,
        grid=(NBLK, NBLK),
        in_specs=[
            pl.BlockSpec((BLK, 3), lambda bi, bj: (bi, 0)),
            pl.BlockSpec((3, BLK), lambda bi, bj: (0, bj)),
            pl.BlockSpec((N_RBF, N_HIDDEN), lambda bi, bj: (0, 0)),
            pl.BlockSpec((1, N_HIDDEN), lambda bi, bj: (0, 0)),
            pl.BlockSpec((N_HIDDEN, N_HIDDEN), lambda bi, bj: (0, 0)),
            pl.BlockSpec((1, N_HIDDEN), lambda bi, bj: (0, 0)),
            pl.BlockSpec((N_HIDDEN, 1), lambda bi, bj: (0, 0)),
            pl.BlockSpec((1, 1), lambda bi, bj: (0, 0)),
            pl.BlockSpec((N_RBF, N_RBF), lambda bi, bj: (0, 0)),
        ],
        out_specs=pl.BlockSpec((1, 1), lambda bi, bj: (0, 0)),
        out_shape=jax.ShapeDtypeStruct((1, 1), jnp.float32),
        scratch_shapes=[
            pltpu.VMEM((N, N_RBF), jnp.float32),
            pltpu.VMEM((N_RBF, N), jnp.float32),
        ],
    )(positions, pos_t, W1, b1r, W2, b2r, W3, b3r, eye)
    return out[0, 0]
